# baseline (speedup 1.0000x reference)
import functools

import jax
import jax.numpy as jnp
from jax import lax
from jax.experimental import pallas as pl
from jax.experimental.pallas import tpu as pltpu
from jax.experimental.pallas import tpu_sc as plsc

_K_RATIO = 0.1
_MIN_K = 5

_NC = 2
_NS = 16
_NW = _NC * _NS
_L = 16
_NB = 256


def _f32_to_key(x):
    bu = lax.bitcast_convert_type(x, jnp.uint32)
    sign = bu >> jnp.uint32(31)
    return bu ^ ((sign * jnp.uint32(0xFFFFFFFF)) | jnp.uint32(0x80000000))


def _key_to_f32(key):
    high = key >= jnp.uint32(0x80000000)
    bu = jnp.where(high, key ^ jnp.uint32(0x80000000), ~key)
    return lax.bitcast_convert_type(bu, jnp.float32)


def _sc_body(x_hbm, out_hbm, xrow, bufa, bufb, hist, red, resv, *, n, k,
             rows_per_w):
    cid = lax.axis_index("c")
    sid = lax.axis_index("s")
    wid = cid * _NS + sid
    lane = lax.iota(jnp.int32, _L)
    lane_base = lane * _NB
    ones = jnp.ones((_L,), jnp.int32)
    kf = jnp.float32(k)

    def zero_hist():
        def body(j, _):
            hist[pl.ds(j * _L, _L)] = jnp.zeros((_L,), jnp.int32)
            return 0
        lax.fori_loop(0, _NB * _L // _L, body, 0)

    def scan_level(r_needed):
        def red_body(j, _):
            acc = hist[pl.ds(j * _L, _L)]
            for l in range(1, _L):
                acc = acc + hist[pl.ds(l * _NB + j * _L, _L)]
            red[pl.ds(j * _L, _L)] = acc
            return 0
        lax.fori_loop(0, _NB // _L, red_body, 0)

        def find_body(jj, carry):
            bbest, acc = carry
            j = _NB // _L - 1 - jj
            chunk = red[pl.ds(j * _L, _L)]
            revc = lax.cumsum(lax.rev(chunk, (0,))) + acc
            bvec = j * _L + (_L - 1) - lane
            cand = jnp.where(revc >= r_needed, bvec, -1)
            return jnp.maximum(bbest, jnp.max(cand)), jnp.max(revc)
        bb, _ = lax.fori_loop(0, _NB // _L, find_body,
                              (jnp.int32(-1), jnp.int32(0)))

        def cnt_body(j, acc):
            chunk = red[pl.ds(j * _L, _L)]
            bvec = j * _L + lane
            return acc + jnp.sum(jnp.where(bvec > bb, chunk, 0))
        cnt_above = lax.fori_loop(0, _NB // _L, cnt_body, jnp.int32(0))
        return bb, cnt_above

    def scatter_level(src, nsurv, shift):
        def body(i, _):
            key = src[pl.ds(i * _L, _L)]
            tail = (i * _L + lane) < nsurv
            b = ((key >> jnp.uint32(shift)) & jnp.uint32(0xFF)).astype(jnp.int32)
            plsc.addupdate_scatter(hist, [lane_base + b], ones, mask=tail)
            return 0
        lax.fori_loop(0, (nsurv + _L - 1) // _L, body, 0)

    def partition(src, dst, nsurv, bb, shift, s_acc):
        def body(i, carry):
            off, s = carry
            key = src[pl.ds(i * _L, _L)]
            tail = (i * _L + lane) < nsurv
            b = ((key >> jnp.uint32(shift)) & jnp.uint32(0xFF)).astype(jnp.int32)
            mgt = tail & (b > bb)
            meq = tail & (b == bb)
            x = _key_to_f32(key)
            s = s + jnp.where(mgt, x, jnp.float32(0.0))
            plsc.store_compressed(dst.at[pl.ds(off, _L)], key, mask=meq)
            return off + jnp.sum(meq.astype(jnp.int32)), s
        return lax.fori_loop(0, (nsurv + _L - 1) // _L, body,
                             (jnp.int32(0), s_acc))

    def row_body(rloc, res_acc):
        row = wid * rows_per_w + rloc
        pltpu.sync_copy(x_hbm.at[pl.ds(row * n, n)], xrow)

        zero_hist()
        def p0_body(i, _):
            x = xrow[pl.ds(i * _L, _L)]
            key = _f32_to_key(x)
            bufa[pl.ds(i * _L, _L)] = key
            b = (key >> jnp.uint32(24)).astype(jnp.int32)
            plsc.addupdate_scatter(hist, [lane_base + b], ones)
            return 0
        lax.fori_loop(0, n // _L, p0_body, 0)

        b1, c1 = scan_level(jnp.int32(k))
        n1, s = partition(bufa, bufb, jnp.int32(n), b1, 24,
                          jnp.zeros((_L,), jnp.float32))
        a = c1

        zero_hist()
        scatter_level(bufb, n1, 16)
        b2, c2 = scan_level(k - a)
        n2, s = partition(bufb, bufa, n1, b2, 16, s)
        a = a + c2

        zero_hist()
        scatter_level(bufa, n2, 8)
        b3, c3 = scan_level(k - a)
        n3, s = partition(bufa, bufb, n2, b3, 8, s)
        a = a + c3

        zero_hist()
        scatter_level(bufb, n3, 0)
        b4, c4 = scan_level(k - a)
        _, s = partition(bufb, bufa, n3, b4, 0, s)
        a = a + c4

        tu = ((b1.astype(jnp.uint32) << jnp.uint32(24))
              | (b2.astype(jnp.uint32) << jnp.uint32(16))
              | (b3.astype(jnp.uint32) << jnp.uint32(8))
              | b4.astype(jnp.uint32))
        tx = _key_to_f32(jnp.zeros((_L,), jnp.uint32) + tu)
        r_v = (jnp.full((_L,), k, jnp.int32) - a).astype(jnp.float32)
        s_tot = jnp.zeros((_L,), jnp.float32) + jnp.sum(s)
        val_v = (s_tot + r_v * tx) / kf
        return jnp.where(lane == rloc, val_v, res_acc)

    res = lax.fori_loop(0, rows_per_w, row_body, jnp.zeros((_L,), jnp.float32))
    resv[...] = res
    pltpu.sync_copy(resv, out_hbm.at[pl.ds(wid * _L, _L)])


def kernel(patch_logits):
    if patch_logits.ndim == 4:
        b = patch_logits.shape[0]
        patch_logits = patch_logits.reshape(b, -1)
    rows, n = patch_logits.shape
    k = max(_MIN_K, int(-(-n * _K_RATIO // 1)))
    rows_per_w = rows // _NW
    mesh = plsc.VectorSubcoreMesh(core_axis_name="c", subcore_axis_name="s",
                                  num_cores=_NC, num_subcores=_NS)
    body = functools.partial(_sc_body, n=n, k=k, rows_per_w=rows_per_w)
    out = pl.kernel(
        body,
        out_type=jax.ShapeDtypeStruct((_NW * _L,), jnp.float32),
        mesh=mesh,
        compiler_params=pltpu.CompilerParams(needs_layout_passes=False),
        scratch_types=[
            pltpu.VMEM((n,), jnp.float32),
            pltpu.VMEM((n + _L,), jnp.uint32),
            pltpu.VMEM((n + _L,), jnp.uint32),
            pltpu.VMEM((_NB * _L,), jnp.int32),
            pltpu.VMEM((_NB,), jnp.int32),
            pltpu.VMEM((_L,), jnp.float32),
        ],
    )(patch_logits.reshape(-1))
    return out.reshape(_NW, _L)[:, :rows_per_w].reshape(rows, 1)
